# baseline (speedup 1.0000x reference)
import functools
import math

import jax
import jax.numpy as jnp
from jax import lax
from jax.experimental import pallas as pl
from jax.experimental.pallas import tpu as pltpu
from jax.experimental.pallas import tpu_sc as plsc

_N = 4096
_V = 32000
_PAD = 0
_SMOOTH = 0.1
_EPS = _SMOOTH / (_V - 2)
_CONF = 1.0 - _SMOOTH
_CF = _CONF - _EPS
_C = _CONF * math.log(_CONF) + _SMOOTH * math.log(_EPS)

_SC_CORES = 2
_SC_SUBCORES = 16
_L = 16
_NW = _SC_CORES * _SC_SUBCORES
_GPW = _N // _NW

_RSC = 4096
_RPT = _RSC // _NW
_CH = _V // 2
_NCH = 2 * _RPT

_BR = 32
_K = 4


def _chunk_sum(buf):
    z = jnp.zeros((_L,), jnp.float32)

    def body(i, accs):
        b = i * (4 * _L)
        return tuple(
            a + buf[pl.ds(b + t * _L, _L)] for t, a in enumerate(accs)
        )

    a0, a1, a2, a3 = lax.fori_loop(0, _CH // (4 * _L), body, (z, z, z, z))
    return (a0 + a1) + (a2 + a3)


def _sc_body(x_hbm, tgt_hbm, o_hbm,
             tgt_v, idx_v, g_v, x0_v, tgtd_v, bufa, bufb, dvec_v, res_v,
             sema, semb):
    wid = lax.axis_index("s") * _SC_CORES + lax.axis_index("c")

    gbase = wid * _GPW
    pltpu.sync_copy(tgt_hbm.at[pl.ds(gbase, _GPW)], tgt_v)

    @pl.loop(0, _GPW, step=_L)
    def _(k):
        rows = (gbase + k) + lax.iota(jnp.int32, _L)
        idx_v[pl.ds(k, _L)] = rows * _V + tgt_v[pl.ds(k, _L)]

    pltpu.async_copy(x_hbm.at[idx_v], g_v, sema).wait()

    @pl.loop(0, _GPW, step=_L)
    def _(k):
        rows = (gbase + k) + lax.iota(jnp.int32, _L)
        idx_v[pl.ds(k, _L)] = rows * _V

    pltpu.async_copy(x_hbm.at[idx_v], x0_v, sema).wait()

    avec = jnp.zeros((_L,), jnp.float32)
    for t in range(_GPW // _L):
        sl = pl.ds(t * _L, _L)
        m = tgt_v[sl] != _PAD
        term = _C + _EPS * x0_v[sl] - _CF * g_v[sl]
        avec = avec + jnp.where(m, term, 0.0)
    res_v[...] = avec

    pltpu.sync_copy(tgt_hbm.at[pl.ds(wid * _RPT, _RPT)],
                    tgtd_v.at[pl.ds(0, _RPT)])
    dvec_v[...] = jnp.zeros((_L,), jnp.float32)
    cbase = wid * _NCH

    def start(c, buf, sem):
        pltpu.make_async_copy(
            x_hbm.at[pl.ds((cbase + c) * _CH, _CH)], buf, sem
        ).start()

    def finish(c, buf, sem):
        pltpu.make_async_copy(
            x_hbm.at[pl.ds((cbase + c) * _CH, _CH)], buf, sem
        ).wait()
        s = _chunk_sum(buf)
        t0 = tgtd_v[pl.ds(c >> 1, _L)][0]
        dvec_v[...] += s * (t0 != _PAD).astype(jnp.float32)

    start(0, bufa, sema)

    @pl.loop(0, _NCH, step=2)
    def _(c):
        start(c + 1, bufb, semb)
        finish(c, bufa, sema)

        @pl.when(c + 2 < _NCH)
        def _():
            start(c + 2, bufa, sema)

        finish(c + 1, bufb, semb)

    res_v[...] = res_v[...] - _EPS * dvec_v[...]
    pltpu.sync_copy(res_v, o_hbm.at[pl.ds(wid * _L, _L)])


def _make_sc_part():
    return pl.kernel(
        _sc_body,
        mesh=plsc.VectorSubcoreMesh(
            core_axis_name="c", subcore_axis_name="s",
            num_cores=_SC_CORES, num_subcores=_SC_SUBCORES,
        ),
        out_type=jax.ShapeDtypeStruct((_NW * _L,), jnp.float32),
        scratch_types=[
            pltpu.VMEM((_GPW,), jnp.int32),
            pltpu.VMEM((_GPW,), jnp.int32),
            pltpu.VMEM((_GPW,), jnp.float32),
            pltpu.VMEM((_GPW,), jnp.float32),
            pltpu.VMEM((_RPT + _L,), jnp.int32),
            pltpu.VMEM((_CH,), jnp.float32),
            pltpu.VMEM((_CH,), jnp.float32),
            pltpu.VMEM((_L,), jnp.float32),
            pltpu.VMEM((_L,), jnp.float32),
            pltpu.SemaphoreType.DMA,
            pltpu.SemaphoreType.DMA,
        ],
    )


def _comb_body(sc_ref, out_ref):
    out_ref[...] = jnp.sum(sc_ref[...]).reshape(1, 1)


def kernel(x, tgt):
    tgt = tgt.astype(jnp.int32)
    sc_part = _make_sc_part()(x.reshape(-1), tgt)
    total = pl.pallas_call(
        _comb_body,
        out_shape=jax.ShapeDtypeStruct((1, 1), jnp.float32),
    )(sc_part.reshape(4, 128))
    return total[0, 0]


# speedup vs baseline: 1.1313x; 1.1313x over previous
import functools
import math

import jax
import jax.numpy as jnp
from jax import lax
from jax.experimental import pallas as pl
from jax.experimental.pallas import tpu as pltpu
from jax.experimental.pallas import tpu_sc as plsc

_N = 4096
_V = 32000
_PAD = 0
_SMOOTH = 0.1
_EPS = _SMOOTH / (_V - 2)
_CONF = 1.0 - _SMOOTH
_CF = _CONF - _EPS
_C = _CONF * math.log(_CONF) + _SMOOTH * math.log(_EPS)

_SC_CORES = 2
_SC_SUBCORES = 16
_L = 16
_NW = _SC_CORES * _SC_SUBCORES
_GPW = _N // _NW

_RSC = 4096
_RPT = _RSC // _NW
_CH = _V
_NCH = _RPT

_BR = 32
_K = 4


def _chunk_sum(buf):
    z = jnp.zeros((_L,), jnp.float32)

    def body(i, accs):
        b = i * (4 * _L)
        return tuple(
            a + buf[pl.ds(b + t * _L, _L)] for t, a in enumerate(accs)
        )

    a0, a1, a2, a3 = lax.fori_loop(0, _CH // (4 * _L), body, (z, z, z, z),
                                   unroll=8)
    return (a0 + a1) + (a2 + a3)


def _sc_body(x_hbm, tgt_hbm, o_hbm,
             tgt_v, idx_v, g_v, x0_v, tgtd_v, bufa, bufb, dvec_v, res_v,
             sema, semb):
    wid = lax.axis_index("s") * _SC_CORES + lax.axis_index("c")

    gbase = wid * _GPW
    pltpu.sync_copy(tgt_hbm.at[pl.ds(gbase, _GPW)], tgt_v)

    @pl.loop(0, _GPW, step=_L)
    def _(k):
        rows = (gbase + k) + lax.iota(jnp.int32, _L)
        idx_v[pl.ds(k, _L)] = rows * _V + tgt_v[pl.ds(k, _L)]

    pltpu.async_copy(x_hbm.at[idx_v], g_v, sema).wait()

    @pl.loop(0, _GPW, step=_L)
    def _(k):
        rows = (gbase + k) + lax.iota(jnp.int32, _L)
        idx_v[pl.ds(k, _L)] = rows * _V

    pltpu.async_copy(x_hbm.at[idx_v], x0_v, sema).wait()

    avec = jnp.zeros((_L,), jnp.float32)
    for t in range(_GPW // _L):
        sl = pl.ds(t * _L, _L)
        m = tgt_v[sl] != _PAD
        term = _C + _EPS * x0_v[sl] - _CF * g_v[sl]
        avec = avec + jnp.where(m, term, 0.0)
    res_v[...] = avec

    pltpu.sync_copy(tgt_hbm.at[pl.ds(wid * _RPT, _RPT)],
                    tgtd_v.at[pl.ds(0, _RPT)])
    dvec_v[...] = jnp.zeros((_L,), jnp.float32)
    cbase = wid * _NCH

    def start(c, buf, sem):
        pltpu.make_async_copy(
            x_hbm.at[pl.ds((cbase + c) * _CH, _CH)], buf, sem
        ).start()

    def finish(c, buf, sem):
        pltpu.make_async_copy(
            x_hbm.at[pl.ds((cbase + c) * _CH, _CH)], buf, sem
        ).wait()
        s = _chunk_sum(buf)
        t0 = tgtd_v[pl.ds(c, _L)][0]
        dvec_v[...] += s * (t0 != _PAD).astype(jnp.float32)

    start(0, bufa, sema)

    @pl.loop(0, _NCH, step=2)
    def _(c):
        start(c + 1, bufb, semb)
        finish(c, bufa, sema)

        @pl.when(c + 2 < _NCH)
        def _():
            start(c + 2, bufa, sema)

        finish(c + 1, bufb, semb)

    res_v[...] = res_v[...] - _EPS * dvec_v[...]
    pltpu.sync_copy(res_v, o_hbm.at[pl.ds(wid * _L, _L)])


def _make_sc_part():
    return pl.kernel(
        _sc_body,
        mesh=plsc.VectorSubcoreMesh(
            core_axis_name="c", subcore_axis_name="s",
            num_cores=_SC_CORES, num_subcores=_SC_SUBCORES,
        ),
        out_type=jax.ShapeDtypeStruct((_NW * _L,), jnp.float32),
        scratch_types=[
            pltpu.VMEM((_GPW,), jnp.int32),
            pltpu.VMEM((_GPW,), jnp.int32),
            pltpu.VMEM((_GPW,), jnp.float32),
            pltpu.VMEM((_GPW,), jnp.float32),
            pltpu.VMEM((_RPT + _L,), jnp.int32),
            pltpu.VMEM((_CH,), jnp.float32),
            pltpu.VMEM((_CH,), jnp.float32),
            pltpu.VMEM((_L,), jnp.float32),
            pltpu.VMEM((_L,), jnp.float32),
            pltpu.SemaphoreType.DMA,
            pltpu.SemaphoreType.DMA,
        ],
    )


def _comb_body(sc_ref, out_ref):
    out_ref[...] = jnp.sum(sc_ref[...]).reshape(1, 1)


def kernel(x, tgt):
    tgt = tgt.astype(jnp.int32)
    sc_part = _make_sc_part()(x.reshape(-1), tgt)
    total = pl.pallas_call(
        _comb_body,
        out_shape=jax.ShapeDtypeStruct((1, 1), jnp.float32),
    )(sc_part.reshape(4, 128))
    return total[0, 0]


# speedup vs baseline: 1.2214x; 1.0796x over previous
import functools
import math

import jax
import jax.numpy as jnp
from jax import lax
from jax.experimental import pallas as pl
from jax.experimental.pallas import tpu as pltpu
from jax.experimental.pallas import tpu_sc as plsc

_N = 4096
_V = 32000
_PAD = 0
_SMOOTH = 0.1
_EPS = _SMOOTH / (_V - 2)
_CONF = 1.0 - _SMOOTH
_CF = _CONF - _EPS
_C = _CONF * math.log(_CONF) + _SMOOTH * math.log(_EPS)

_SC_CORES = 2
_SC_SUBCORES = 16
_L = 16
_NW = _SC_CORES * _SC_SUBCORES
_GPW = _N // _NW

_RSC = 2048
_RPT = _RSC // _NW
_CH = _V
_NCH = _RPT

_BR = 32
_K = 4


def _chunk_sum(buf):
    z = jnp.zeros((_L,), jnp.float32)

    def body(i, accs):
        b = i * (4 * _L)
        return tuple(
            a + buf[pl.ds(b + t * _L, _L)] for t, a in enumerate(accs)
        )

    a0, a1, a2, a3 = lax.fori_loop(0, _CH // (4 * _L), body, (z, z, z, z),
                                   unroll=8)
    return (a0 + a1) + (a2 + a3)


def _sc_body(x_hbm, tgt_hbm, o_hbm,
             tgt_v, idx_v, g_v, x0_v, tgtd_v, bufa, bufb, dvec_v, res_v,
             sema, semb):
    wid = lax.axis_index("s") * _SC_CORES + lax.axis_index("c")

    gbase = wid * _GPW
    pltpu.sync_copy(tgt_hbm.at[pl.ds(gbase, _GPW)], tgt_v)

    @pl.loop(0, _GPW, step=_L)
    def _(k):
        rows = (gbase + k) + lax.iota(jnp.int32, _L)
        idx_v[pl.ds(k, _L)] = rows * _V + tgt_v[pl.ds(k, _L)]

    pltpu.async_copy(x_hbm.at[idx_v], g_v, sema).wait()

    @pl.loop(0, _GPW, step=_L)
    def _(k):
        rows = (gbase + k) + lax.iota(jnp.int32, _L)
        idx_v[pl.ds(k, _L)] = rows * _V

    pltpu.async_copy(x_hbm.at[idx_v], x0_v, sema).wait()

    avec = jnp.zeros((_L,), jnp.float32)
    for t in range(_GPW // _L):
        sl = pl.ds(t * _L, _L)
        m = tgt_v[sl] != _PAD
        term = _C + _EPS * x0_v[sl] - _CF * g_v[sl]
        avec = avec + jnp.where(m, term, 0.0)
    res_v[...] = avec

    pltpu.sync_copy(tgt_hbm.at[pl.ds(wid * _RPT, _RPT)],
                    tgtd_v.at[pl.ds(0, _RPT)])
    dvec_v[...] = jnp.zeros((_L,), jnp.float32)
    cbase = wid * _NCH

    def start(c, buf, sem):
        pltpu.make_async_copy(
            x_hbm.at[pl.ds((cbase + c) * _CH, _CH)], buf, sem
        ).start()

    def finish(c, buf, sem):
        pltpu.make_async_copy(
            x_hbm.at[pl.ds((cbase + c) * _CH, _CH)], buf, sem
        ).wait()
        s = _chunk_sum(buf)
        t0 = tgtd_v[pl.ds(c, _L)][0]
        dvec_v[...] += s * (t0 != _PAD).astype(jnp.float32)

    start(0, bufa, sema)

    @pl.loop(0, _NCH, step=2)
    def _(c):
        start(c + 1, bufb, semb)
        finish(c, bufa, sema)

        @pl.when(c + 2 < _NCH)
        def _():
            start(c + 2, bufa, sema)

        finish(c + 1, bufb, semb)

    res_v[...] = res_v[...] - _EPS * dvec_v[...]
    pltpu.sync_copy(res_v, o_hbm.at[pl.ds(wid * _L, _L)])


def _make_sc_part():
    return pl.kernel(
        _sc_body,
        mesh=plsc.VectorSubcoreMesh(
            core_axis_name="c", subcore_axis_name="s",
            num_cores=_SC_CORES, num_subcores=_SC_SUBCORES,
        ),
        out_type=jax.ShapeDtypeStruct((_NW * _L,), jnp.float32),
        scratch_types=[
            pltpu.VMEM((_GPW,), jnp.int32),
            pltpu.VMEM((_GPW,), jnp.int32),
            pltpu.VMEM((_GPW,), jnp.float32),
            pltpu.VMEM((_GPW,), jnp.float32),
            pltpu.VMEM((_RPT + _L,), jnp.int32),
            pltpu.VMEM((_CH,), jnp.float32),
            pltpu.VMEM((_CH,), jnp.float32),
            pltpu.VMEM((_L,), jnp.float32),
            pltpu.VMEM((_L,), jnp.float32),
            pltpu.SemaphoreType.DMA,
            pltpu.SemaphoreType.DMA,
        ],
    )


def _tc_body(tgt_ref, *refs):
    *x_refs, out_ref = refs
    j = pl.program_id(0)

    @pl.when(j == 0)
    def _():
        out_ref[...] = jnp.zeros((1, 1), jnp.float32)

    nonpad = (tgt_ref[...] != _PAD).astype(jnp.float32)
    acc = jnp.zeros((1, 1), jnp.float32)
    for k, x_ref in enumerate(x_refs):
        sl = slice(k * _BR, (k + 1) * _BR)
        rowsums = jnp.sum(x_ref[...], axis=1, keepdims=True)
        acc += jnp.sum(nonpad[sl, :] * rowsums).reshape(1, 1)
    out_ref[...] += -_EPS * acc


def _comb_body(sc_ref, t_ref, out_ref):
    out_ref[...] = (jnp.sum(sc_ref[...]) + jnp.sum(t_ref[...])).reshape(1, 1)


def kernel(x, tgt):
    tgt = tgt.astype(jnp.int32)
    sc_part = _make_sc_part()(x.reshape(-1), tgt)

    row0 = _RSC // _BR
    x_specs = [
        pl.BlockSpec(
            (_BR, _V),
            functools.partial(lambda k, j: (row0 + j * _K + k, 0), k),
        )
        for k in range(_K)
    ]
    tc_part = pl.pallas_call(
        _tc_body,
        grid=((_N - _RSC) // (_BR * _K),),
        in_specs=[
            pl.BlockSpec((_K * _BR, 1), lambda j: (_RSC // (_K * _BR) + j, 0)),
            *x_specs,
        ],
        out_specs=pl.BlockSpec((1, 1), lambda j: (0, 0)),
        out_shape=jax.ShapeDtypeStruct((1, 1), jnp.float32),
    )(tgt.reshape(_N, 1), *([x] * _K))

    total = pl.pallas_call(
        _comb_body,
        out_shape=jax.ShapeDtypeStruct((1, 1), jnp.float32),
    )(sc_part.reshape(4, 128), tc_part)
    return total[0, 0]


# speedup vs baseline: 1.2410x; 1.0161x over previous
import functools
import math

import jax
import jax.numpy as jnp
from jax import lax
from jax.experimental import pallas as pl
from jax.experimental.pallas import tpu as pltpu
from jax.experimental.pallas import tpu_sc as plsc

_N = 4096
_V = 32000
_PAD = 0
_SMOOTH = 0.1
_EPS = _SMOOTH / (_V - 2)
_CONF = 1.0 - _SMOOTH
_C = _CONF * math.log(_CONF) + _SMOOTH * math.log(_EPS)

_SC_CORES = 2
_SC_SUBCORES = 16
_SC_LANES = 16
_NW = _SC_CORES * _SC_SUBCORES
_BPW = _N // _NW

_BR = 32
_K = 4


def _sc_gather_body(x_hbm, tgt_hbm, g_hbm, tgt_v, idx_v, g_v, sem):
    wid = lax.axis_index("s") * _SC_CORES + lax.axis_index("c")
    base = wid * _BPW
    pltpu.sync_copy(tgt_hbm.at[pl.ds(base, _BPW)], tgt_v)

    @pl.loop(0, _BPW, step=_SC_LANES)
    def _(k):
        rows = (base + k) + lax.iota(jnp.int32, _SC_LANES)
        idx_v[pl.ds(k, _SC_LANES)] = rows * _V + tgt_v[pl.ds(k, _SC_LANES)]

    pltpu.async_copy(x_hbm.at[idx_v], g_v, sem).wait()
    pltpu.sync_copy(g_v, g_hbm.at[pl.ds(base, _BPW)])


def _tc_body(tgt_ref, g_ref, *refs):
    *x_refs, out_ref = refs
    j = pl.program_id(0)

    @pl.when(j == 0)
    def _():
        out_ref[...] = jnp.zeros((1, 1), jnp.float32)

    nonpad = (tgt_ref[...] != _PAD).astype(jnp.float32)
    acc = jnp.zeros((1, 1), jnp.float32)
    for k, x_ref in enumerate(x_refs):
        sl = slice(k * _BR, (k + 1) * _BR)
        rowsums = jnp.sum(x_ref[...], axis=1, keepdims=True)
        x0 = x_ref[:, 0:1]
        per_row = (_C + _EPS * x0 - (_CONF - _EPS) * g_ref[sl, :]
                   - _EPS * rowsums)
        acc += jnp.sum(nonpad[sl, :] * per_row).reshape(1, 1)
    out_ref[...] += acc


def _make_sc_gather():
    return pl.kernel(
        _sc_gather_body,
        mesh=plsc.VectorSubcoreMesh(
            core_axis_name="c", subcore_axis_name="s",
            num_cores=_SC_CORES, num_subcores=_SC_SUBCORES,
        ),
        out_type=jax.ShapeDtypeStruct((_N,), jnp.float32),
        scratch_types=[
            pltpu.VMEM((_BPW,), jnp.int32),
            pltpu.VMEM((_BPW,), jnp.int32),
            pltpu.VMEM((_BPW,), jnp.float32),
            pltpu.SemaphoreType.DMA,
        ],
    )


def kernel(x, tgt):
    tgt = tgt.astype(jnp.int32)
    g = _make_sc_gather()(x.reshape(-1), tgt)
    x_specs = [
        pl.BlockSpec((_BR, _V), functools.partial(lambda k, j: (j * _K + k, 0), k))
        for k in range(_K)
    ]
    total = pl.pallas_call(
        _tc_body,
        grid=(_N // (_BR * _K),),
        in_specs=[
            pl.BlockSpec((_K * _BR, 1), lambda j: (j, 0)),
            pl.BlockSpec((_K * _BR, 1), lambda j: (j, 0)),
            *x_specs,
        ],
        out_specs=pl.BlockSpec((1, 1), lambda j: (0, 0)),
        out_shape=jax.ShapeDtypeStruct((1, 1), jnp.float32),
    )(tgt.reshape(_N, 1), g.reshape(_N, 1), *([x] * _K))
    return total[0, 0]
